# Initial kernel scaffold; baseline (speedup 1.0000x reference)
import jax
import jax.numpy as jnp
from jax.experimental import pallas as pl


def kernel(added_points, original_points, boxes, ego_loc):
    raise NotImplementedError("write your pallas kernel here")


# speedup vs baseline: 13.1617x; 13.1617x over previous
import functools

import jax
import jax.numpy as jnp
from jax import lax
from jax.experimental import pallas as pl
from jax.experimental.pallas import tpu as pltpu
from jax.experimental.pallas import tpu_sc as plsc

H = W = 256
L = 16
NS = 16
ROWS = H // NS
NCH = W // L
T = 50
HALO = 8
SROWS = ROWS + 1
PB = 6


def _body(added, original, scal, xrow_rep, krow_rep, nrow_rep, ycol, kcol,
          ncol, parts, out,
          ch_v, scal_v, xrow_v, krow_v, nrow_v, ycol_v, kcol_v, ncol_v,
          inbox_v, mo_v, mp_v, go_v, gp_v, parti_v, partu_v, redall_v,
          iou_v, sem):
    c = lax.axis_index("c")
    s = lax.axis_index("s")
    row0 = s * ROWS

    zeros = jnp.zeros((L,), jnp.float32)
    ones = jnp.full((L,), 1.0, dtype=jnp.float32)

    cps = []
    for k in range(4):
        cps.append(pltpu.async_copy(
            added.at[c, k, pl.ds(row0, ROWS)], ch_v.at[k, pl.ds(0, ROWS)], sem))
    for k in range(4):
        cps.append(pltpu.async_copy(
            original.at[c, k + 1, pl.ds(row0, ROWS)],
            ch_v.at[4 + k, pl.ds(0, ROWS)], sem))
    cps.append(pltpu.async_copy(scal.at[pl.ds(c * (T * PB * L), T * PB * L)],
                                scal_v, sem))
    cps.append(pltpu.async_copy(xrow_rep.at[pl.ds(row0 * L, SROWS * L)],
                                xrow_v, sem))
    cps.append(pltpu.async_copy(krow_rep.at[pl.ds(row0 * L, SROWS * L)],
                                krow_v, sem))
    cps.append(pltpu.async_copy(nrow_rep.at[pl.ds(row0 * L, SROWS * L)],
                                nrow_v, sem))
    cps.append(pltpu.async_copy(ycol, ycol_v, sem))
    cps.append(pltpu.async_copy(kcol, kcol_v, sem))
    cps.append(pltpu.async_copy(ncol, ncol_v, sem))

    @pl.when(s < NS - 1)
    def _():
        hcps = []
        for k in range(4):
            hcps.append(pltpu.async_copy(
                added.at[c, k, pl.ds(row0 + ROWS, HALO)],
                ch_v.at[k, pl.ds(ROWS, HALO)], sem))
        for k in range(4):
            hcps.append(pltpu.async_copy(
                original.at[c, k + 1, pl.ds(row0 + ROWS, HALO)],
                ch_v.at[4 + k, pl.ds(ROWS, HALO)], sem))
        for cp in hcps:
            cp.wait()

    @pl.when(s == NS - 1)
    def _():
        for k in range(8):
            for j in range(NCH):
                ch_v[k, ROWS, pl.ds(j * L, L)] = zeros

    for cp in cps:
        cp.wait()

    def mask_body(r, _):
        for j in range(NCH):
            sl = pl.ds(j * L, L)
            sp = ch_v[0, r, sl] + ch_v[1, r, sl] + ch_v[2, r, sl] + ch_v[3, r, sl]
            so = ch_v[4, r, sl] + ch_v[5, r, sl] + ch_v[6, r, sl] + ch_v[7, r, sl]
            mp_v[r, sl] = jnp.where(sp != 0.0, ones, zeros)
            mo_v[r, sl] = jnp.where(so != 0.0, ones, zeros)
            inbox_v[r, sl] = zeros
        return 0
    lax.fori_loop(0, SROWS, mask_body, 0)

    def box_body(t, _):
        base = t * (PB * L)
        cxv = scal_v[pl.ds(base, L)]
        cyv = scal_v[pl.ds(base + L, L)]
        cav = scal_v[pl.ds(base + 2 * L, L)]
        sav = scal_v[pl.ds(base + 3 * L, L)]
        hxv = scal_v[pl.ds(base + 4 * L, L)]
        hyv = scal_v[pl.ds(base + 5 * L, L)]
        for r in range(SROWS):
            sx = xrow_v[pl.ds(r * L, L)] - cxv
            av = sx * cav
            bv = sx * sav
            for j in range(NCH):
                sl = pl.ds(j * L, L)
                sy = ycol_v[sl] - cyv
                lx = av - sy * sav
                ly = bv + sy * cav
                tb = (jnp.abs(lx) < hxv) & (jnp.abs(ly) < hyv)
                inbox_v[r, sl] = jnp.maximum(
                    inbox_v[r, sl], jnp.where(tb, ones, zeros))
        return 0
    lax.fori_loop(0, T, box_body, 0)

    go_v[pl.ds(W, L)] = zeros
    gp_v[pl.ds(W, L)] = zeros

    def fold_body(r, carry):
        acc_i, acc_u = carry
        kr0 = krow_v[pl.ds(r * L, L)]
        nr1 = nrow_v[pl.ds(r * L + L, L)]
        for j in range(NCH):
            sl = pl.ds(j * L, L)
            ib0 = inbox_v[r, sl]
            ib1 = inbox_v[r + 1, sl]
            go_v[sl] = jnp.maximum(ib0 * mo_v[r, sl] * kr0,
                                   ib1 * mo_v[r + 1, sl] * nr1)
            gp_v[sl] = jnp.maximum(ib0 * mp_v[r, sl] * kr0,
                                   ib1 * mp_v[r + 1, sl] * nr1)
        for j in range(NCH):
            sl = pl.ds(j * L, L)
            sl1 = pl.ds(j * L + 1, L)
            g0 = jnp.maximum(go_v[sl] * kcol_v[sl], go_v[sl1] * ncol_v[sl1])
            g1 = jnp.maximum(gp_v[sl] * kcol_v[sl], gp_v[sl1] * ncol_v[sl1])
            acc_i = acc_i + plsc.all_reduce_population_count(g0 * g1 != 0.0)
            acc_u = acc_u + plsc.all_reduce_population_count(
                jnp.maximum(g0, g1) != 0.0)
        return acc_i, acc_u

    izeros = jnp.zeros((L,), jnp.int32)
    acc_i, acc_u = lax.fori_loop(0, ROWS, fold_body, (izeros, izeros))

    parti_v[...] = acc_i.astype(jnp.float32)
    partu_v[...] = acc_u.astype(jnp.float32)
    base = (c * NS + s) * 2 * L
    pltpu.sync_copy(parti_v, parts.at[pl.ds(base, L)])
    pltpu.sync_copy(partu_v, parts.at[pl.ds(base + L, L)])
    plsc.subcore_barrier()

    @pl.when(s == 0)
    def _():
        pltpu.sync_copy(parts.at[pl.ds(c * NS * 2 * L, NS * 2 * L)], redall_v)
        ti = redall_v[pl.ds(0, L)]
        tu = redall_v[pl.ds(L, L)]
        for k in range(1, NS):
            ti = ti + redall_v[pl.ds(k * 2 * L, L)]
            tu = tu + redall_v[pl.ds(k * 2 * L + L, L)]
        iou_v[...] = ti / jnp.maximum(tu, ones)
        pltpu.sync_copy(iou_v, out.at[pl.ds(c * L, L)])


@jax.jit
def kernel(added_points, original_points, boxes, ego_loc):
    B = added_points.shape[0]
    f32 = jnp.float32

    r = jnp.arange(256, dtype=f32)
    v = (r - 128.0) * 0.8
    m = (lax.optimization_barrier(v) / 0.8 + 128.0).astype(jnp.int32)
    keep = (m == jnp.arange(256)).astype(f32)
    notk = 1.0 - keep

    cxs = boxes[:, :, 0] + (-ego_loc[:, 0:1])
    cys = boxes[:, :, 1] + (-ego_loc[:, 1:2])
    nrz = -boxes[:, :, 6]
    ca = jnp.cos(nrz)
    sa = jnp.sin(nrz)
    zok = jnp.abs(f32(0.8) - boxes[:, :, 2]) < boxes[:, :, 5] * 0.5
    hx = jnp.where(zok, boxes[:, :, 3] * 0.5, f32(-1.0))
    hy = boxes[:, :, 4] * 0.5
    scal = jnp.stack([cxs, cys, ca, sa, hx, hy], axis=2)
    scal_rep = jnp.broadcast_to(
        scal[..., None], (B, T, PB, L)).astype(f32).reshape(-1)

    pad1 = jnp.zeros((1,), f32)
    xrow_rep = jnp.broadcast_to(
        jnp.concatenate([v, pad1])[:, None], (257, L)).reshape(-1)
    krow_rep = jnp.broadcast_to(
        jnp.concatenate([keep, pad1])[:, None], (257, L)).reshape(-1)
    nrow_rep = jnp.broadcast_to(
        jnp.concatenate([notk, pad1])[:, None], (257, L)).reshape(-1)
    ycol = v
    kcol = jnp.concatenate([keep, jnp.zeros((L,), f32)])
    ncol = jnp.concatenate([notk, jnp.zeros((L,), f32)])

    mesh = plsc.VectorSubcoreMesh(core_axis_name="c", subcore_axis_name="s")
    run = functools.partial(
        pl.kernel,
        mesh=mesh,
        compiler_params=pltpu.CompilerParams(needs_layout_passes=False),
        out_type=[jax.ShapeDtypeStruct((B * NS * 2 * L,), f32),
                  jax.ShapeDtypeStruct((B * L,), f32)],
        scratch_types=[
            pltpu.VMEM((8, ROWS + HALO, W), f32),
            pltpu.VMEM((T * PB * L,), f32),
            pltpu.VMEM((SROWS * L,), f32),
            pltpu.VMEM((SROWS * L,), f32),
            pltpu.VMEM((SROWS * L,), f32),
            pltpu.VMEM((W,), f32),
            pltpu.VMEM((W + L,), f32),
            pltpu.VMEM((W + L,), f32),
            pltpu.VMEM((SROWS, W), f32),
            pltpu.VMEM((SROWS, W), f32),
            pltpu.VMEM((SROWS, W), f32),
            pltpu.VMEM((W + L,), f32),
            pltpu.VMEM((W + L,), f32),
            pltpu.VMEM((L,), f32),
            pltpu.VMEM((L,), f32),
            pltpu.VMEM((NS * 2 * L,), f32),
            pltpu.VMEM((L,), f32),
            pltpu.SemaphoreType.DMA,
        ],
    )(_body)
    _, out = run(added_points, original_points, scal_rep, xrow_rep, krow_rep,
                 nrow_rep, ycol, kcol, ncol)
    return jnp.mean(out.reshape(B, L)[:, 0])


# speedup vs baseline: 20.2697x; 1.5400x over previous
import functools

import jax
import jax.numpy as jnp
from jax import lax
from jax.experimental import pallas as pl
from jax.experimental.pallas import tpu as pltpu
from jax.experimental.pallas import tpu_sc as plsc

H = W = 256
L = 16
NS = 16
ROWS = H // NS
NCH = W // L
T = 50
HALO = 8
SROWS = ROWS + 1
PB = 6


def _body(added, original, scal, bounds, xrow_rep, krow_rep, nrow_rep, ycol,
          kcol, ncol, parts, out,
          ch_v, scal_v, bounds_v, xrow_v, krow_v, nrow_v, ycol_v, kcol_v,
          ncol_v, inbox_v, mo_v, mp_v, go_v, gp_v, parti_v, partu_v, redall_v,
          iou_v, sem):
    c = lax.axis_index("c")
    s = lax.axis_index("s")
    row0 = s * ROWS

    zeros = jnp.zeros((L,), jnp.float32)
    ones = jnp.full((L,), 1.0, dtype=jnp.float32)

    cps = []
    for k in range(4):
        cps.append(pltpu.async_copy(
            added.at[c, k, pl.ds(row0, ROWS)], ch_v.at[k, pl.ds(0, ROWS)], sem))
    for k in range(4):
        cps.append(pltpu.async_copy(
            original.at[c, k + 1, pl.ds(row0, ROWS)],
            ch_v.at[4 + k, pl.ds(0, ROWS)], sem))
    cps.append(pltpu.async_copy(scal.at[pl.ds(c * (T * PB * L), T * PB * L)],
                                scal_v, sem))
    cps.append(pltpu.async_copy(bounds.at[pl.ds(c * (T * L), T * L)],
                                bounds_v, sem))
    cps.append(pltpu.async_copy(xrow_rep.at[pl.ds(row0 * L, SROWS * L)],
                                xrow_v, sem))
    cps.append(pltpu.async_copy(krow_rep.at[pl.ds(row0 * L, SROWS * L)],
                                krow_v, sem))
    cps.append(pltpu.async_copy(nrow_rep.at[pl.ds(row0 * L, SROWS * L)],
                                nrow_v, sem))
    cps.append(pltpu.async_copy(ycol, ycol_v, sem))
    cps.append(pltpu.async_copy(kcol, kcol_v, sem))
    cps.append(pltpu.async_copy(ncol, ncol_v, sem))

    @pl.when(s < NS - 1)
    def _():
        hcps = []
        for k in range(4):
            hcps.append(pltpu.async_copy(
                added.at[c, k, pl.ds(row0 + ROWS, HALO)],
                ch_v.at[k, pl.ds(ROWS, HALO)], sem))
        for k in range(4):
            hcps.append(pltpu.async_copy(
                original.at[c, k + 1, pl.ds(row0 + ROWS, HALO)],
                ch_v.at[4 + k, pl.ds(ROWS, HALO)], sem))
        for cp in hcps:
            cp.wait()

    @pl.when(s == NS - 1)
    def _():
        for k in range(8):
            for j in range(NCH):
                ch_v[k, ROWS, pl.ds(j * L, L)] = zeros

    for cp in cps:
        cp.wait()

    def mask_body(r, _):
        for j in range(NCH):
            sl = pl.ds(j * L, L)
            sp = ch_v[0, r, sl] + ch_v[1, r, sl] + ch_v[2, r, sl] + ch_v[3, r, sl]
            so = ch_v[4, r, sl] + ch_v[5, r, sl] + ch_v[6, r, sl] + ch_v[7, r, sl]
            mp_v[r, sl] = jnp.where(sp != 0.0, ones, zeros)
            mo_v[r, sl] = jnp.where(so != 0.0, ones, zeros)
            inbox_v[r, sl] = zeros
        return 0
    lax.fori_loop(0, SROWS, mask_body, 0)

    def box_body(t, _):
        base = t * (PB * L)
        cxv = scal_v[pl.ds(base, L)]
        cyv = scal_v[pl.ds(base + L, L)]
        cav = scal_v[pl.ds(base + 2 * L, L)]
        sav = scal_v[pl.ds(base + 3 * L, L)]
        hxv = scal_v[pl.ds(base + 4 * L, L)]
        hyv = scal_v[pl.ds(base + 5 * L, L)]
        bvec = bounds_v[pl.ds(t * L, L)]
        rlo = jnp.maximum(bvec[0] - row0, 0)
        rhi = jnp.minimum(bvec[1] - row0, SROWS)
        jlo = bvec[2]
        jhi = bvec[3]

        def row_body(r, _):
            sx = xrow_v[pl.ds(r * L, L)] - cxv
            av = sx * cav
            bv = sx * sav

            def col_body(j, _):
                sl = pl.ds(j * L, L)
                sy = ycol_v[sl] - cyv
                lx = av - sy * sav
                ly = bv + sy * cav
                tb = (jnp.abs(lx) < hxv) & (jnp.abs(ly) < hyv)
                inbox_v[r, sl] = jnp.maximum(
                    inbox_v[r, sl], jnp.where(tb, ones, zeros))
                return 0
            lax.fori_loop(jlo, jhi, col_body, 0)
            return 0
        lax.fori_loop(rlo, rhi, row_body, 0)
        return 0
    lax.fori_loop(0, T, box_body, 0)

    go_v[pl.ds(W, L)] = zeros
    gp_v[pl.ds(W, L)] = zeros

    def fold_body(r, carry):
        acc_i, acc_u = carry
        kr0 = krow_v[pl.ds(r * L, L)]
        nr1 = nrow_v[pl.ds(r * L + L, L)]
        for j in range(NCH):
            sl = pl.ds(j * L, L)
            ib0 = inbox_v[r, sl]
            ib1 = inbox_v[r + 1, sl]
            go_v[sl] = jnp.maximum(ib0 * mo_v[r, sl] * kr0,
                                   ib1 * mo_v[r + 1, sl] * nr1)
            gp_v[sl] = jnp.maximum(ib0 * mp_v[r, sl] * kr0,
                                   ib1 * mp_v[r + 1, sl] * nr1)
        for j in range(NCH):
            sl = pl.ds(j * L, L)
            sl1 = pl.ds(j * L + 1, L)
            g0 = jnp.maximum(go_v[sl] * kcol_v[sl], go_v[sl1] * ncol_v[sl1])
            g1 = jnp.maximum(gp_v[sl] * kcol_v[sl], gp_v[sl1] * ncol_v[sl1])
            acc_i = acc_i + plsc.all_reduce_population_count(g0 * g1 != 0.0)
            acc_u = acc_u + plsc.all_reduce_population_count(
                jnp.maximum(g0, g1) != 0.0)
        return acc_i, acc_u

    izeros = jnp.zeros((L,), jnp.int32)
    acc_i, acc_u = lax.fori_loop(0, ROWS, fold_body, (izeros, izeros))

    parti_v[...] = acc_i.astype(jnp.float32)
    partu_v[...] = acc_u.astype(jnp.float32)
    base = (c * NS + s) * 2 * L
    pltpu.sync_copy(parti_v, parts.at[pl.ds(base, L)])
    pltpu.sync_copy(partu_v, parts.at[pl.ds(base + L, L)])
    plsc.subcore_barrier()

    @pl.when(s == 0)
    def _():
        pltpu.sync_copy(parts.at[pl.ds(c * NS * 2 * L, NS * 2 * L)], redall_v)
        ti = redall_v[pl.ds(0, L)]
        tu = redall_v[pl.ds(L, L)]
        for k in range(1, NS):
            ti = ti + redall_v[pl.ds(k * 2 * L, L)]
            tu = tu + redall_v[pl.ds(k * 2 * L + L, L)]
        iou_v[...] = ti / jnp.maximum(tu, ones)
        pltpu.sync_copy(iou_v, out.at[pl.ds(c * L, L)])


@jax.jit
def kernel(added_points, original_points, boxes, ego_loc):
    B = added_points.shape[0]
    f32 = jnp.float32

    r = jnp.arange(256, dtype=f32)
    v = (r - 128.0) * 0.8
    m = (lax.optimization_barrier(v) / 0.8 + 128.0).astype(jnp.int32)
    keep = (m == jnp.arange(256)).astype(f32)
    notk = 1.0 - keep

    cxs = boxes[:, :, 0] + (-ego_loc[:, 0:1])
    cys = boxes[:, :, 1] + (-ego_loc[:, 1:2])
    nrz = -boxes[:, :, 6]
    ca = jnp.cos(nrz)
    sa = jnp.sin(nrz)
    zok = jnp.abs(f32(0.8) - boxes[:, :, 2]) < boxes[:, :, 5] * 0.5
    hx = jnp.where(zok, boxes[:, :, 3] * 0.5, f32(-1.0))
    hy = boxes[:, :, 4] * 0.5
    scal = jnp.stack([cxs, cys, ca, sa, hx, hy], axis=2)
    scal_rep = jnp.broadcast_to(
        scal[..., None], (B, T, PB, L)).astype(f32).reshape(-1)

    rad = jnp.sqrt(boxes[:, :, 3] ** 2 + boxes[:, :, 4] ** 2) * 0.5

    def _bnd(center, radius):
        lo = jnp.floor((center - radius) / 0.8 + 128.0) - 1.0
        hi = jnp.ceil((center + radius) / 0.8 + 128.0) + 2.0
        lo = jnp.clip(jnp.nan_to_num(lo, nan=0.0, posinf=256.0, neginf=0.0),
                      0.0, 256.0)
        hi = jnp.clip(jnp.nan_to_num(hi, nan=256.0, posinf=256.0, neginf=0.0),
                      0.0, 256.0)
        return lo.astype(jnp.int32), hi.astype(jnp.int32)

    rlo, rhi = _bnd(cxs, rad)
    clo, chi = _bnd(cys, rad)
    jlo = clo // L
    jhi = (chi + L - 1) // L
    bounds = jnp.concatenate(
        [jnp.stack([rlo, rhi, jlo, jhi], axis=2),
         jnp.zeros((B, T, L - 4), jnp.int32)], axis=2).reshape(-1)

    pad1 = jnp.zeros((1,), f32)
    xrow_rep = jnp.broadcast_to(
        jnp.concatenate([v, pad1])[:, None], (257, L)).reshape(-1)
    krow_rep = jnp.broadcast_to(
        jnp.concatenate([keep, pad1])[:, None], (257, L)).reshape(-1)
    nrow_rep = jnp.broadcast_to(
        jnp.concatenate([notk, pad1])[:, None], (257, L)).reshape(-1)
    ycol = v
    kcol = jnp.concatenate([keep, jnp.zeros((L,), f32)])
    ncol = jnp.concatenate([notk, jnp.zeros((L,), f32)])

    mesh = plsc.VectorSubcoreMesh(core_axis_name="c", subcore_axis_name="s")
    run = functools.partial(
        pl.kernel,
        mesh=mesh,
        compiler_params=pltpu.CompilerParams(needs_layout_passes=False),
        out_type=[jax.ShapeDtypeStruct((B * NS * 2 * L,), f32),
                  jax.ShapeDtypeStruct((B * L,), f32)],
        scratch_types=[
            pltpu.VMEM((8, ROWS + HALO, W), f32),
            pltpu.VMEM((T * PB * L,), f32),
            pltpu.VMEM((T * L,), jnp.int32),
            pltpu.VMEM((SROWS * L,), f32),
            pltpu.VMEM((SROWS * L,), f32),
            pltpu.VMEM((SROWS * L,), f32),
            pltpu.VMEM((W,), f32),
            pltpu.VMEM((W + L,), f32),
            pltpu.VMEM((W + L,), f32),
            pltpu.VMEM((SROWS, W), f32),
            pltpu.VMEM((SROWS, W), f32),
            pltpu.VMEM((SROWS, W), f32),
            pltpu.VMEM((W + L,), f32),
            pltpu.VMEM((W + L,), f32),
            pltpu.VMEM((L,), f32),
            pltpu.VMEM((L,), f32),
            pltpu.VMEM((NS * 2 * L,), f32),
            pltpu.VMEM((L,), f32),
            pltpu.SemaphoreType.DMA,
        ],
    )(_body)
    _, out = run(added_points, original_points, scal_rep, bounds, xrow_rep,
                 krow_rep, nrow_rep, ycol, kcol, ncol)
    return jnp.mean(out.reshape(B, L)[:, 0])


# speedup vs baseline: 20.3420x; 1.0036x over previous
import functools

import jax
import jax.numpy as jnp
from jax import lax
from jax.experimental import pallas as pl
from jax.experimental.pallas import tpu as pltpu
from jax.experimental.pallas import tpu_sc as plsc

H = W = 256
L = 16
NS = 16
ROWS = H // NS
NCH = W // L
T = 50
HALO = 8
SROWS = ROWS + 1
PB = 6


def _body(added, original, scal, bounds, xrow_rep, krow_rep, nrow_rep, ycol,
          kcol, ncol, parts, out,
          ch_v, scal_v, bounds_v, xrow_v, krow_v, nrow_v, ycol_v, kcol_v,
          ncol_v, inbox_v, mo_v, mp_v, go_v, gp_v, parti_v, partu_v, redall_v,
          iou_v, sem):
    c = lax.axis_index("c")
    s = lax.axis_index("s")
    row0 = s * ROWS

    zeros = jnp.zeros((L,), jnp.float32)
    ones = jnp.full((L,), 1.0, dtype=jnp.float32)

    cps = []
    for k in range(4):
        cps.append(pltpu.async_copy(
            added.at[c, k, pl.ds(row0, ROWS)], ch_v.at[k, pl.ds(0, ROWS)], sem))
    for k in range(4):
        cps.append(pltpu.async_copy(
            original.at[c, k + 1, pl.ds(row0, ROWS)],
            ch_v.at[4 + k, pl.ds(0, ROWS)], sem))
    cps.append(pltpu.async_copy(scal.at[pl.ds(c * (T * PB * L), T * PB * L)],
                                scal_v, sem))
    cps.append(pltpu.async_copy(bounds.at[pl.ds(c * (T * L), T * L)],
                                bounds_v, sem))
    cps.append(pltpu.async_copy(xrow_rep.at[pl.ds(row0 * L, SROWS * L)],
                                xrow_v, sem))
    cps.append(pltpu.async_copy(krow_rep.at[pl.ds(row0 * L, SROWS * L)],
                                krow_v, sem))
    cps.append(pltpu.async_copy(nrow_rep.at[pl.ds(row0 * L, SROWS * L)],
                                nrow_v, sem))
    cps.append(pltpu.async_copy(ycol, ycol_v, sem))
    cps.append(pltpu.async_copy(kcol, kcol_v, sem))
    cps.append(pltpu.async_copy(ncol, ncol_v, sem))

    @pl.when(s < NS - 1)
    def _():
        hcps = []
        for k in range(4):
            hcps.append(pltpu.async_copy(
                added.at[c, k, pl.ds(row0 + ROWS, HALO)],
                ch_v.at[k, pl.ds(ROWS, HALO)], sem))
        for k in range(4):
            hcps.append(pltpu.async_copy(
                original.at[c, k + 1, pl.ds(row0 + ROWS, HALO)],
                ch_v.at[4 + k, pl.ds(ROWS, HALO)], sem))
        for cp in hcps:
            cp.wait()

    @pl.when(s == NS - 1)
    def _():
        for k in range(8):
            for j in range(NCH):
                ch_v[k, ROWS, pl.ds(j * L, L)] = zeros

    for cp in cps:
        cp.wait()

    def mask_body(r, _):
        for j in range(NCH):
            sl = pl.ds(j * L, L)
            sp = ch_v[0, r, sl] + ch_v[1, r, sl] + ch_v[2, r, sl] + ch_v[3, r, sl]
            so = ch_v[4, r, sl] + ch_v[5, r, sl] + ch_v[6, r, sl] + ch_v[7, r, sl]
            mp_v[r, sl] = jnp.where(sp != 0.0, ones, zeros)
            mo_v[r, sl] = jnp.where(so != 0.0, ones, zeros)
            inbox_v[r, sl] = zeros
        return 0
    lax.fori_loop(0, SROWS, mask_body, 0)

    def box_body(t, _):
        base = t * (PB * L)
        cxv = scal_v[pl.ds(base, L)]
        cyv = scal_v[pl.ds(base + L, L)]
        cav = scal_v[pl.ds(base + 2 * L, L)]
        sav = scal_v[pl.ds(base + 3 * L, L)]
        hxv = scal_v[pl.ds(base + 4 * L, L)]
        hyv = scal_v[pl.ds(base + 5 * L, L)]
        bvec = bounds_v[pl.ds(t * L, L)]
        rlo = jnp.maximum(bvec[0] - row0, 0)
        rhi = jnp.minimum(bvec[1] - row0, SROWS)
        jlo = bvec[2]
        jhi = bvec[3]

        def row_body(r, _):
            sx = xrow_v[pl.ds(r * L, L)] - cxv
            av = sx * cav
            bv = sx * sav

            def col_body(j, _):
                sl = pl.ds(j * L, L)
                sy = ycol_v[sl] - cyv
                lx = av - sy * sav
                ly = bv + sy * cav
                tb = (jnp.abs(lx) < hxv) & (jnp.abs(ly) < hyv)
                inbox_v[r, sl] = jnp.maximum(
                    inbox_v[r, sl], jnp.where(tb, ones, zeros))
                return 0
            lax.fori_loop(jlo, jhi, col_body, 0)
            return 0
        lax.fori_loop(rlo, rhi, row_body, 0)
        return 0
    lax.fori_loop(0, T, box_body, 0)

    go_v[pl.ds(W, L)] = zeros
    gp_v[pl.ds(W, L)] = zeros

    def fold_body(r, carry):
        acc_i, acc_u = carry
        kr0 = krow_v[pl.ds(r * L, L)]
        nr1 = nrow_v[pl.ds(r * L + L, L)]
        for j in range(NCH):
            sl = pl.ds(j * L, L)
            t0 = inbox_v[r, sl] * kr0
            t1 = inbox_v[r + 1, sl] * nr1
            go_v[sl] = jnp.maximum(t0 * mo_v[r, sl], t1 * mo_v[r + 1, sl])
            gp_v[sl] = jnp.maximum(t0 * mp_v[r, sl], t1 * mp_v[r + 1, sl])
        for j in range(NCH):
            sl = pl.ds(j * L, L)
            sl1 = pl.ds(j * L + 1, L)
            g0 = jnp.maximum(go_v[sl] * kcol_v[sl], go_v[sl1] * ncol_v[sl1])
            g1 = jnp.maximum(gp_v[sl] * kcol_v[sl], gp_v[sl1] * ncol_v[sl1])
            acc_i = acc_i + plsc.all_reduce_population_count(g0 * g1 != 0.0)
            acc_u = acc_u + plsc.all_reduce_population_count(
                jnp.maximum(g0, g1) != 0.0)
        return acc_i, acc_u

    izeros = jnp.zeros((L,), jnp.int32)
    acc_i, acc_u = lax.fori_loop(0, ROWS, fold_body, (izeros, izeros))

    parti_v[...] = acc_i.astype(jnp.float32)
    partu_v[...] = acc_u.astype(jnp.float32)
    base = (c * NS + s) * 2 * L
    pltpu.sync_copy(parti_v, parts.at[pl.ds(base, L)])
    pltpu.sync_copy(partu_v, parts.at[pl.ds(base + L, L)])
    plsc.subcore_barrier()

    @pl.when(s == 0)
    def _():
        pltpu.sync_copy(parts.at[pl.ds(c * NS * 2 * L, NS * 2 * L)], redall_v)
        ti = redall_v[pl.ds(0, L)]
        tu = redall_v[pl.ds(L, L)]
        for k in range(1, NS):
            ti = ti + redall_v[pl.ds(k * 2 * L, L)]
            tu = tu + redall_v[pl.ds(k * 2 * L + L, L)]
        iou_v[...] = ti / jnp.maximum(tu, ones)
        pltpu.sync_copy(iou_v, out.at[pl.ds(c * L, L)])


@jax.jit
def kernel(added_points, original_points, boxes, ego_loc):
    B = added_points.shape[0]
    f32 = jnp.float32

    r = jnp.arange(256, dtype=f32)
    v = (r - 128.0) * 0.8
    m = (lax.optimization_barrier(v) / 0.8 + 128.0).astype(jnp.int32)
    keep = (m == jnp.arange(256)).astype(f32)
    notk = 1.0 - keep

    cxs = boxes[:, :, 0] + (-ego_loc[:, 0:1])
    cys = boxes[:, :, 1] + (-ego_loc[:, 1:2])
    nrz = -boxes[:, :, 6]
    ca = jnp.cos(nrz)
    sa = jnp.sin(nrz)
    zok = jnp.abs(f32(0.8) - boxes[:, :, 2]) < boxes[:, :, 5] * 0.5
    hx = jnp.where(zok, boxes[:, :, 3] * 0.5, f32(-1.0))
    hy = boxes[:, :, 4] * 0.5
    scal = jnp.stack([cxs, cys, ca, sa, hx, hy], axis=2)
    scal_rep = jnp.broadcast_to(
        scal[..., None], (B, T, PB, L)).astype(f32).reshape(-1)

    rad = jnp.sqrt(boxes[:, :, 3] ** 2 + boxes[:, :, 4] ** 2) * 0.5

    def _bnd(center, radius):
        lo = jnp.floor((center - radius) / 0.8 + 128.0) - 1.0
        hi = jnp.ceil((center + radius) / 0.8 + 128.0) + 2.0
        lo = jnp.clip(jnp.nan_to_num(lo, nan=0.0, posinf=256.0, neginf=0.0),
                      0.0, 256.0)
        hi = jnp.clip(jnp.nan_to_num(hi, nan=256.0, posinf=256.0, neginf=0.0),
                      0.0, 256.0)
        return lo.astype(jnp.int32), hi.astype(jnp.int32)

    rlo, rhi = _bnd(cxs, rad)
    clo, chi = _bnd(cys, rad)
    jlo = clo // L
    jhi = (chi + L - 1) // L
    bounds = jnp.concatenate(
        [jnp.stack([rlo, rhi, jlo, jhi], axis=2),
         jnp.zeros((B, T, L - 4), jnp.int32)], axis=2).reshape(-1)

    pad1 = jnp.zeros((1,), f32)
    xrow_rep = jnp.broadcast_to(
        jnp.concatenate([v, pad1])[:, None], (257, L)).reshape(-1)
    krow_rep = jnp.broadcast_to(
        jnp.concatenate([keep, pad1])[:, None], (257, L)).reshape(-1)
    nrow_rep = jnp.broadcast_to(
        jnp.concatenate([notk, pad1])[:, None], (257, L)).reshape(-1)
    ycol = v
    kcol = jnp.concatenate([keep, jnp.zeros((L,), f32)])
    ncol = jnp.concatenate([notk, jnp.zeros((L,), f32)])

    mesh = plsc.VectorSubcoreMesh(core_axis_name="c", subcore_axis_name="s")
    run = functools.partial(
        pl.kernel,
        mesh=mesh,
        compiler_params=pltpu.CompilerParams(needs_layout_passes=False),
        out_type=[jax.ShapeDtypeStruct((B * NS * 2 * L,), f32),
                  jax.ShapeDtypeStruct((B * L,), f32)],
        scratch_types=[
            pltpu.VMEM((8, ROWS + HALO, W), f32),
            pltpu.VMEM((T * PB * L,), f32),
            pltpu.VMEM((T * L,), jnp.int32),
            pltpu.VMEM((SROWS * L,), f32),
            pltpu.VMEM((SROWS * L,), f32),
            pltpu.VMEM((SROWS * L,), f32),
            pltpu.VMEM((W,), f32),
            pltpu.VMEM((W + L,), f32),
            pltpu.VMEM((W + L,), f32),
            pltpu.VMEM((SROWS, W), f32),
            pltpu.VMEM((SROWS, W), f32),
            pltpu.VMEM((SROWS, W), f32),
            pltpu.VMEM((W + L,), f32),
            pltpu.VMEM((W + L,), f32),
            pltpu.VMEM((L,), f32),
            pltpu.VMEM((L,), f32),
            pltpu.VMEM((NS * 2 * L,), f32),
            pltpu.VMEM((L,), f32),
            pltpu.SemaphoreType.DMA,
        ],
    )(_body)
    _, out = run(added_points, original_points, scal_rep, bounds, xrow_rep,
                 krow_rep, nrow_rep, ycol, kcol, ncol)
    return jnp.mean(out.reshape(B, L)[:, 0])


# speedup vs baseline: 22.1278x; 1.0878x over previous
import functools

import jax
import jax.numpy as jnp
from jax import lax
from jax.experimental import pallas as pl
from jax.experimental.pallas import tpu as pltpu
from jax.experimental.pallas import tpu_sc as plsc

H = W = 256
L = 16
NS = 16
ROWS = H // NS
NCH = W // L
T = 50
HALO = 8
SROWS = ROWS + 1
PB = 6


OFF_SCAL = 0
OFF_XROW = OFF_SCAL + 2 * T * PB * L
OFF_KROW = OFF_XROW + 257 * L
OFF_NROW = OFF_KROW + 257 * L
OFF_YCOL = OFF_NROW + 257 * L
OFF_KCOL = OFF_YCOL + W
OFF_NCOL = OFF_KCOL + W + L
OFF_BND = OFF_NCOL + W + L
AUX_LEN = OFF_BND + 2 * T * L


def _body(added, original, aux, parts, out,
          ch_v, scal_v, bounds_v, xrow_v, krow_v, nrow_v, ycol_v, kcol_v,
          ncol_v, inbox_v, mo_v, mp_v, go_v, gp_v, parti_v, partu_v, redall_v,
          iou_v, sem):
    c = lax.axis_index("c")
    s = lax.axis_index("s")
    row0 = s * ROWS

    zeros = jnp.zeros((L,), jnp.float32)
    ones = jnp.full((L,), 1.0, dtype=jnp.float32)

    cps = []
    for k in range(4):
        cps.append(pltpu.async_copy(
            added.at[c, k, pl.ds(row0, ROWS)], ch_v.at[k, pl.ds(0, ROWS)], sem))
    for k in range(4):
        cps.append(pltpu.async_copy(
            original.at[c, k + 1, pl.ds(row0, ROWS)],
            ch_v.at[4 + k, pl.ds(0, ROWS)], sem))
    cps.append(pltpu.async_copy(
        aux.at[pl.ds(OFF_SCAL + c * (T * PB * L), T * PB * L)], scal_v, sem))
    cps.append(pltpu.async_copy(
        aux.at[pl.ds(OFF_BND + c * (T * L), T * L)], bounds_v, sem))
    cps.append(pltpu.async_copy(
        aux.at[pl.ds(OFF_XROW + row0 * L, SROWS * L)], xrow_v, sem))
    cps.append(pltpu.async_copy(
        aux.at[pl.ds(OFF_KROW + row0 * L, SROWS * L)], krow_v, sem))
    cps.append(pltpu.async_copy(
        aux.at[pl.ds(OFF_NROW + row0 * L, SROWS * L)], nrow_v, sem))
    cps.append(pltpu.async_copy(aux.at[pl.ds(OFF_YCOL, W)], ycol_v, sem))
    cps.append(pltpu.async_copy(aux.at[pl.ds(OFF_KCOL, W + L)], kcol_v, sem))
    cps.append(pltpu.async_copy(aux.at[pl.ds(OFF_NCOL, W + L)], ncol_v, sem))

    @pl.when(s < NS - 1)
    def _():
        hcps = []
        for k in range(4):
            hcps.append(pltpu.async_copy(
                added.at[c, k, pl.ds(row0 + ROWS, HALO)],
                ch_v.at[k, pl.ds(ROWS, HALO)], sem))
        for k in range(4):
            hcps.append(pltpu.async_copy(
                original.at[c, k + 1, pl.ds(row0 + ROWS, HALO)],
                ch_v.at[4 + k, pl.ds(ROWS, HALO)], sem))
        for cp in hcps:
            cp.wait()

    @pl.when(s == NS - 1)
    def _():
        for k in range(8):
            for j in range(NCH):
                ch_v[k, ROWS, pl.ds(j * L, L)] = zeros

    for cp in cps:
        cp.wait()

    def mask_body(r, _):
        for j in range(NCH):
            sl = pl.ds(j * L, L)
            sp = ch_v[0, r, sl] + ch_v[1, r, sl] + ch_v[2, r, sl] + ch_v[3, r, sl]
            so = ch_v[4, r, sl] + ch_v[5, r, sl] + ch_v[6, r, sl] + ch_v[7, r, sl]
            mp_v[r, sl] = jnp.where(sp != 0.0, ones, zeros)
            mo_v[r, sl] = jnp.where(so != 0.0, ones, zeros)
            inbox_v[r, sl] = zeros
        return 0
    lax.fori_loop(0, SROWS, mask_body, 0)

    def box_body(t, _):
        base = t * (PB * L)
        cxv = scal_v[pl.ds(base, L)]
        cyv = scal_v[pl.ds(base + L, L)]
        cav = scal_v[pl.ds(base + 2 * L, L)]
        sav = scal_v[pl.ds(base + 3 * L, L)]
        hxv = scal_v[pl.ds(base + 4 * L, L)]
        hyv = scal_v[pl.ds(base + 5 * L, L)]
        bvec = plsc.bitcast(bounds_v[pl.ds(t * L, L)], jnp.int32)
        rlo = jnp.maximum(bvec[0] - row0, 0)
        rhi = jnp.minimum(bvec[1] - row0, SROWS)
        jlo = bvec[2]
        jhi = bvec[3]

        def row_body(r, _):
            sx = xrow_v[pl.ds(r * L, L)] - cxv
            av = sx * cav
            bv = sx * sav

            def col_body(j, _):
                sl = pl.ds(j * L, L)
                sy = ycol_v[sl] - cyv
                lx = av - sy * sav
                ly = bv + sy * cav
                tb = (jnp.abs(lx) < hxv) & (jnp.abs(ly) < hyv)
                inbox_v[r, sl] = jnp.maximum(
                    inbox_v[r, sl], jnp.where(tb, ones, zeros))
                return 0
            lax.fori_loop(jlo, jhi, col_body, 0)
            return 0
        lax.fori_loop(rlo, rhi, row_body, 0)
        return 0
    lax.fori_loop(0, T, box_body, 0)

    go_v[pl.ds(W, L)] = zeros
    gp_v[pl.ds(W, L)] = zeros

    def fold_body(r, carry):
        acc_i, acc_u = carry
        kr0 = krow_v[pl.ds(r * L, L)]
        nr1 = nrow_v[pl.ds(r * L + L, L)]
        for j in range(NCH):
            sl = pl.ds(j * L, L)
            t0 = inbox_v[r, sl] * kr0
            t1 = inbox_v[r + 1, sl] * nr1
            go_v[sl] = jnp.maximum(t0 * mo_v[r, sl], t1 * mo_v[r + 1, sl])
            gp_v[sl] = jnp.maximum(t0 * mp_v[r, sl], t1 * mp_v[r + 1, sl])
        for j in range(NCH):
            sl = pl.ds(j * L, L)
            sl1 = pl.ds(j * L + 1, L)
            g0 = jnp.maximum(go_v[sl] * kcol_v[sl], go_v[sl1] * ncol_v[sl1])
            g1 = jnp.maximum(gp_v[sl] * kcol_v[sl], gp_v[sl1] * ncol_v[sl1])
            acc_i = acc_i + plsc.all_reduce_population_count(g0 * g1 != 0.0)
            acc_u = acc_u + plsc.all_reduce_population_count(
                jnp.maximum(g0, g1) != 0.0)
        return acc_i, acc_u

    izeros = jnp.zeros((L,), jnp.int32)
    acc_i, acc_u = lax.fori_loop(0, ROWS, fold_body, (izeros, izeros))

    parti_v[...] = acc_i.astype(jnp.float32)
    partu_v[...] = acc_u.astype(jnp.float32)
    base = (c * NS + s) * 2 * L
    pltpu.sync_copy(parti_v, parts.at[pl.ds(base, L)])
    pltpu.sync_copy(partu_v, parts.at[pl.ds(base + L, L)])
    plsc.subcore_barrier()

    @pl.when(s == 0)
    def _():
        pltpu.sync_copy(parts.at[pl.ds(c * NS * 2 * L, NS * 2 * L)], redall_v)
        ti = redall_v[pl.ds(0, L)]
        tu = redall_v[pl.ds(L, L)]
        for k in range(1, NS):
            ti = ti + redall_v[pl.ds(k * 2 * L, L)]
            tu = tu + redall_v[pl.ds(k * 2 * L + L, L)]
        iou_v[...] = ti / jnp.maximum(tu, ones)
        pltpu.sync_copy(iou_v, out.at[pl.ds(c * L, L)])


@jax.jit
def kernel(added_points, original_points, boxes, ego_loc):
    B = added_points.shape[0]
    f32 = jnp.float32

    r = jnp.arange(256, dtype=f32)
    v = (r - 128.0) * 0.8
    m = (lax.optimization_barrier(v) / 0.8 + 128.0).astype(jnp.int32)
    keep = (m == jnp.arange(256)).astype(f32)
    notk = 1.0 - keep

    cxs = boxes[:, :, 0] + (-ego_loc[:, 0:1])
    cys = boxes[:, :, 1] + (-ego_loc[:, 1:2])
    nrz = -boxes[:, :, 6]
    ca = jnp.cos(nrz)
    sa = jnp.sin(nrz)
    zok = jnp.abs(f32(0.8) - boxes[:, :, 2]) < boxes[:, :, 5] * 0.5
    hx = jnp.where(zok, boxes[:, :, 3] * 0.5, f32(-1.0))
    hy = boxes[:, :, 4] * 0.5
    scal = jnp.stack([cxs, cys, ca, sa, hx, hy], axis=2)
    scal_rep = jnp.broadcast_to(
        scal[..., None], (B, T, PB, L)).astype(f32).reshape(-1)

    rad = jnp.sqrt(boxes[:, :, 3] ** 2 + boxes[:, :, 4] ** 2) * 0.5

    def _bnd(center, radius):
        lo = jnp.floor((center - radius) / 0.8 + 128.0) - 1.0
        hi = jnp.ceil((center + radius) / 0.8 + 128.0) + 2.0
        lo = jnp.clip(jnp.nan_to_num(lo, nan=0.0, posinf=256.0, neginf=0.0),
                      0.0, 256.0)
        hi = jnp.clip(jnp.nan_to_num(hi, nan=256.0, posinf=256.0, neginf=0.0),
                      0.0, 256.0)
        return lo.astype(jnp.int32), hi.astype(jnp.int32)

    rlo, rhi = _bnd(cxs, rad)
    clo, chi = _bnd(cys, rad)
    jlo = clo // L
    jhi = (chi + L - 1) // L
    bounds = jnp.concatenate(
        [jnp.stack([rlo, rhi, jlo, jhi], axis=2),
         jnp.zeros((B, T, L - 4), jnp.int32)], axis=2).reshape(-1)

    pad1 = jnp.zeros((1,), f32)
    xrow_rep = jnp.broadcast_to(
        jnp.concatenate([v, pad1])[:, None], (257, L)).reshape(-1)
    krow_rep = jnp.broadcast_to(
        jnp.concatenate([keep, pad1])[:, None], (257, L)).reshape(-1)
    nrow_rep = jnp.broadcast_to(
        jnp.concatenate([notk, pad1])[:, None], (257, L)).reshape(-1)
    ycol = v
    kcol = jnp.concatenate([keep, jnp.zeros((L,), f32)])
    ncol = jnp.concatenate([notk, jnp.zeros((L,), f32)])
    aux = jnp.concatenate([
        scal_rep, xrow_rep, krow_rep, nrow_rep, ycol, kcol, ncol,
        lax.bitcast_convert_type(bounds, f32)])

    mesh = plsc.VectorSubcoreMesh(core_axis_name="c", subcore_axis_name="s")
    run = functools.partial(
        pl.kernel,
        mesh=mesh,
        compiler_params=pltpu.CompilerParams(needs_layout_passes=False),
        out_type=[jax.ShapeDtypeStruct((B * NS * 2 * L,), f32),
                  jax.ShapeDtypeStruct((B * L,), f32)],
        scratch_types=[
            pltpu.VMEM((8, ROWS + HALO, W), f32),
            pltpu.VMEM((T * PB * L,), f32),
            pltpu.VMEM((T * L,), f32),
            pltpu.VMEM((SROWS * L,), f32),
            pltpu.VMEM((SROWS * L,), f32),
            pltpu.VMEM((SROWS * L,), f32),
            pltpu.VMEM((W,), f32),
            pltpu.VMEM((W + L,), f32),
            pltpu.VMEM((W + L,), f32),
            pltpu.VMEM((SROWS, W), f32),
            pltpu.VMEM((SROWS, W), f32),
            pltpu.VMEM((SROWS, W), f32),
            pltpu.VMEM((W + L,), f32),
            pltpu.VMEM((W + L,), f32),
            pltpu.VMEM((L,), f32),
            pltpu.VMEM((L,), f32),
            pltpu.VMEM((NS * 2 * L,), f32),
            pltpu.VMEM((L,), f32),
            pltpu.SemaphoreType.DMA,
        ],
    )(_body)
    _, out = run(added_points, original_points, aux)
    return jnp.mean(out.reshape(B, L)[:, 0])


# speedup vs baseline: 22.1364x; 1.0004x over previous
import functools

import jax
import jax.numpy as jnp
from jax import lax
from jax.experimental import pallas as pl
from jax.experimental.pallas import tpu as pltpu
from jax.experimental.pallas import tpu_sc as plsc

H = W = 256
L = 16
NS = 16
ROWS = H // NS
NCH = W // L
T = 50
HALO = 8
SROWS = ROWS + 1
PB = 6


OFF_SCAL = 0
OFF_XROW = OFF_SCAL + 2 * T * PB * L
OFF_KROW = OFF_XROW + 257 * L
OFF_NROW = OFF_KROW + 257 * L
OFF_YCOL = OFF_NROW + 257 * L
OFF_KCOL = OFF_YCOL + W
OFF_NCOL = OFF_KCOL + W + L
OFF_BND = OFF_NCOL + W + L
AUX_LEN = OFF_BND + 2 * T * L


def _body(added, original, aux, out,
          ch_v, scal_v, bounds_v, xrow_v, krow_v, nrow_v, ycol_v, kcol_v,
          ncol_v, inbox_v, mo_v, mp_v, go_v, gp_v, parti_v, partu_v, redall_v,
          iou_v, sem):
    c = lax.axis_index("c")
    s = lax.axis_index("s")
    row0 = s * ROWS

    zeros = jnp.zeros((L,), jnp.float32)
    ones = jnp.full((L,), 1.0, dtype=jnp.float32)

    cps = []
    for k in range(4):
        cps.append(pltpu.async_copy(
            added.at[c, k, pl.ds(row0, ROWS)], ch_v.at[k, pl.ds(0, ROWS)], sem))
    for k in range(4):
        cps.append(pltpu.async_copy(
            original.at[c, k + 1, pl.ds(row0, ROWS)],
            ch_v.at[4 + k, pl.ds(0, ROWS)], sem))
    cps.append(pltpu.async_copy(
        aux.at[pl.ds(OFF_SCAL + c * (T * PB * L), T * PB * L)], scal_v, sem))
    cps.append(pltpu.async_copy(
        aux.at[pl.ds(OFF_BND + c * (T * L), T * L)], bounds_v, sem))
    cps.append(pltpu.async_copy(
        aux.at[pl.ds(OFF_XROW + row0 * L, SROWS * L)], xrow_v, sem))
    cps.append(pltpu.async_copy(
        aux.at[pl.ds(OFF_KROW + row0 * L, SROWS * L)], krow_v, sem))
    cps.append(pltpu.async_copy(
        aux.at[pl.ds(OFF_NROW + row0 * L, SROWS * L)], nrow_v, sem))
    cps.append(pltpu.async_copy(aux.at[pl.ds(OFF_YCOL, W)], ycol_v, sem))
    cps.append(pltpu.async_copy(aux.at[pl.ds(OFF_KCOL, W + L)], kcol_v, sem))
    cps.append(pltpu.async_copy(aux.at[pl.ds(OFF_NCOL, W + L)], ncol_v, sem))

    @pl.when(s < NS - 1)
    def _():
        hcps = []
        for k in range(4):
            hcps.append(pltpu.async_copy(
                added.at[c, k, pl.ds(row0 + ROWS, HALO)],
                ch_v.at[k, pl.ds(ROWS, HALO)], sem))
        for k in range(4):
            hcps.append(pltpu.async_copy(
                original.at[c, k + 1, pl.ds(row0 + ROWS, HALO)],
                ch_v.at[4 + k, pl.ds(ROWS, HALO)], sem))
        for cp in hcps:
            cp.wait()

    @pl.when(s == NS - 1)
    def _():
        for k in range(8):
            for j in range(NCH):
                ch_v[k, ROWS, pl.ds(j * L, L)] = zeros

    for cp in cps:
        cp.wait()

    def mask_body(r, _):
        for j in range(NCH):
            sl = pl.ds(j * L, L)
            sp = ch_v[0, r, sl] + ch_v[1, r, sl] + ch_v[2, r, sl] + ch_v[3, r, sl]
            so = ch_v[4, r, sl] + ch_v[5, r, sl] + ch_v[6, r, sl] + ch_v[7, r, sl]
            mp_v[r, sl] = jnp.where(sp != 0.0, ones, zeros)
            mo_v[r, sl] = jnp.where(so != 0.0, ones, zeros)
            inbox_v[r, sl] = zeros
        return 0
    lax.fori_loop(0, SROWS, mask_body, 0)

    def box_body(t, _):
        base = t * (PB * L)
        cxv = scal_v[pl.ds(base, L)]
        cyv = scal_v[pl.ds(base + L, L)]
        cav = scal_v[pl.ds(base + 2 * L, L)]
        sav = scal_v[pl.ds(base + 3 * L, L)]
        hxv = scal_v[pl.ds(base + 4 * L, L)]
        hyv = scal_v[pl.ds(base + 5 * L, L)]
        bvec = plsc.bitcast(bounds_v[pl.ds(t * L, L)], jnp.int32)
        rlo = jnp.maximum(bvec[0] - row0, 0)
        rhi = jnp.minimum(bvec[1] - row0, SROWS)
        jlo = bvec[2]
        jhi = bvec[3]

        def row_body(r, _):
            sx = xrow_v[pl.ds(r * L, L)] - cxv
            av = sx * cav
            bv = sx * sav

            def col_body(j, _):
                sl = pl.ds(j * L, L)
                sy = ycol_v[sl] - cyv
                lx = av - sy * sav
                ly = bv + sy * cav
                tb = (jnp.abs(lx) < hxv) & (jnp.abs(ly) < hyv)
                inbox_v[r, sl] = jnp.maximum(
                    inbox_v[r, sl], jnp.where(tb, ones, zeros))
                return 0
            lax.fori_loop(jlo, jhi, col_body, 0)
            return 0
        lax.fori_loop(rlo, rhi, row_body, 0)
        return 0
    lax.fori_loop(0, T, box_body, 0)

    go_v[pl.ds(W, L)] = zeros
    gp_v[pl.ds(W, L)] = zeros

    def fold_body(r, carry):
        acc_i, acc_u = carry
        kr0 = krow_v[pl.ds(r * L, L)]
        nr1 = nrow_v[pl.ds(r * L + L, L)]
        for j in range(NCH):
            sl = pl.ds(j * L, L)
            t0 = inbox_v[r, sl] * kr0
            t1 = inbox_v[r + 1, sl] * nr1
            go_v[sl] = jnp.maximum(t0 * mo_v[r, sl], t1 * mo_v[r + 1, sl])
            gp_v[sl] = jnp.maximum(t0 * mp_v[r, sl], t1 * mp_v[r + 1, sl])
        for j in range(NCH):
            sl = pl.ds(j * L, L)
            sl1 = pl.ds(j * L + 1, L)
            g0 = jnp.maximum(go_v[sl] * kcol_v[sl], go_v[sl1] * ncol_v[sl1])
            g1 = jnp.maximum(gp_v[sl] * kcol_v[sl], gp_v[sl1] * ncol_v[sl1])
            acc_i = acc_i + plsc.all_reduce_population_count(g0 * g1 != 0.0)
            acc_u = acc_u + plsc.all_reduce_population_count(
                jnp.maximum(g0, g1) != 0.0)
        return acc_i, acc_u

    izeros = jnp.zeros((L,), jnp.int32)
    acc_i, acc_u = lax.fori_loop(0, ROWS, fold_body, (izeros, izeros))

    parti_v[...] = acc_i.astype(jnp.float32)
    partu_v[...] = acc_u.astype(jnp.float32)
    base = 2 * L + (c * NS + s) * 2 * L
    pltpu.sync_copy(parti_v, out.at[pl.ds(base, L)])
    pltpu.sync_copy(partu_v, out.at[pl.ds(base + L, L)])
    plsc.subcore_barrier()

    @pl.when(s == 0)
    def _():
        pltpu.sync_copy(out.at[pl.ds(2 * L + c * NS * 2 * L, NS * 2 * L)],
                        redall_v)
        ti = redall_v[pl.ds(0, L)]
        tu = redall_v[pl.ds(L, L)]
        for k in range(1, NS):
            ti = ti + redall_v[pl.ds(k * 2 * L, L)]
            tu = tu + redall_v[pl.ds(k * 2 * L + L, L)]
        iou_v[...] = ti / jnp.maximum(tu, ones)
        pltpu.sync_copy(iou_v, out.at[pl.ds(c * L, L)])


@jax.jit
def kernel(added_points, original_points, boxes, ego_loc):
    B = added_points.shape[0]
    f32 = jnp.float32

    r = jnp.arange(256, dtype=f32)
    v = (r - 128.0) * 0.8
    m = (lax.optimization_barrier(v) / 0.8 + 128.0).astype(jnp.int32)
    keep = (m == jnp.arange(256)).astype(f32)
    notk = 1.0 - keep

    cxs = boxes[:, :, 0] + (-ego_loc[:, 0:1])
    cys = boxes[:, :, 1] + (-ego_loc[:, 1:2])
    nrz = -boxes[:, :, 6]
    ca = jnp.cos(nrz)
    sa = jnp.sin(nrz)
    zok = jnp.abs(f32(0.8) - boxes[:, :, 2]) < boxes[:, :, 5] * 0.5
    hx = jnp.where(zok, boxes[:, :, 3] * 0.5, f32(-1.0))
    hy = boxes[:, :, 4] * 0.5
    scal = jnp.stack([cxs, cys, ca, sa, hx, hy], axis=2)
    scal_rep = jnp.broadcast_to(
        scal[..., None], (B, T, PB, L)).astype(f32).reshape(-1)

    rad = jnp.sqrt(boxes[:, :, 3] ** 2 + boxes[:, :, 4] ** 2) * 0.5

    def _bnd(center, radius):
        lo = jnp.floor((center - radius) / 0.8 + 128.0) - 1.0
        hi = jnp.ceil((center + radius) / 0.8 + 128.0) + 2.0
        lo = jnp.clip(jnp.nan_to_num(lo, nan=0.0, posinf=256.0, neginf=0.0),
                      0.0, 256.0)
        hi = jnp.clip(jnp.nan_to_num(hi, nan=256.0, posinf=256.0, neginf=0.0),
                      0.0, 256.0)
        return lo.astype(jnp.int32), hi.astype(jnp.int32)

    rlo, rhi = _bnd(cxs, rad)
    clo, chi = _bnd(cys, rad)
    jlo = clo // L
    jhi = (chi + L - 1) // L
    bounds = jnp.concatenate(
        [jnp.stack([rlo, rhi, jlo, jhi], axis=2),
         jnp.zeros((B, T, L - 4), jnp.int32)], axis=2).reshape(-1)

    pad1 = jnp.zeros((1,), f32)
    xrow_rep = jnp.broadcast_to(
        jnp.concatenate([v, pad1])[:, None], (257, L)).reshape(-1)
    krow_rep = jnp.broadcast_to(
        jnp.concatenate([keep, pad1])[:, None], (257, L)).reshape(-1)
    nrow_rep = jnp.broadcast_to(
        jnp.concatenate([notk, pad1])[:, None], (257, L)).reshape(-1)
    ycol = v
    kcol = jnp.concatenate([keep, jnp.zeros((L,), f32)])
    ncol = jnp.concatenate([notk, jnp.zeros((L,), f32)])
    aux = jnp.concatenate([
        scal_rep, xrow_rep, krow_rep, nrow_rep, ycol, kcol, ncol,
        lax.bitcast_convert_type(bounds, f32)])

    mesh = plsc.VectorSubcoreMesh(core_axis_name="c", subcore_axis_name="s")
    run = functools.partial(
        pl.kernel,
        mesh=mesh,
        compiler_params=pltpu.CompilerParams(needs_layout_passes=False),
        out_type=jax.ShapeDtypeStruct((B * L + B * NS * 2 * L,), f32),
        scratch_types=[
            pltpu.VMEM((8, ROWS + HALO, W), f32),
            pltpu.VMEM((T * PB * L,), f32),
            pltpu.VMEM((T * L,), f32),
            pltpu.VMEM((SROWS * L,), f32),
            pltpu.VMEM((SROWS * L,), f32),
            pltpu.VMEM((SROWS * L,), f32),
            pltpu.VMEM((W,), f32),
            pltpu.VMEM((W + L,), f32),
            pltpu.VMEM((W + L,), f32),
            pltpu.VMEM((SROWS, W), f32),
            pltpu.VMEM((SROWS, W), f32),
            pltpu.VMEM((SROWS, W), f32),
            pltpu.VMEM((W + L,), f32),
            pltpu.VMEM((W + L,), f32),
            pltpu.VMEM((L,), f32),
            pltpu.VMEM((L,), f32),
            pltpu.VMEM((NS * 2 * L,), f32),
            pltpu.VMEM((L,), f32),
            pltpu.SemaphoreType.DMA,
        ],
    )(_body)
    out = run(added_points, original_points, aux)
    return jnp.mean(out[:B * L].reshape(B, L)[:, 0])
